# baseline (speedup 1.0000x reference)
import functools

import jax
import jax.numpy as jnp
from jax import lax
from jax.experimental import pallas as pl
from jax.experimental.pallas import tpu as pltpu
from jax.experimental.pallas import tpu_sc as plsc

N = 10000
E = 320000
D = 128
C1 = 100
C2 = 10
THR = 1.0 / (C1 - 1)
EPS = 1e-5

NC = 2
NS = 16
KCH = 128
G = 8
NB0 = 4
NB1 = 16
TB = NS * (NB0 + NB1)
EPAD = TB * G * KCH
NP = 10240
RT = NP // NS

RB = 1000
GB = N // RB


def _sc_pass_body(table, idxg, idxs, zeros, out, idxg_v, idxs_v, rows_v, acc,
                  semg0, semg1, sems0, sems1, semr0, semr1):
    c = lax.axis_index("c")
    s = lax.axis_index("s")
    nb = jnp.where(c == 0, NB0, NB1)
    b0 = c * NS * NB0 + s * nb
    semg = (semg0, semg1)
    semsd = (sems0, sems1)
    semr = (semr0, semr1)

    def fire_idx(slot, b):
        pltpu.async_copy(idxg.at[b0 + b], idxg_v.at[pl.ds(slot * G, G)], semg[slot])
        pltpu.async_copy(idxs.at[b0 + b], idxs_v.at[pl.ds(slot * G, G)], semsd[slot])

    def wait_idx(slot, b):
        pltpu.make_async_copy(idxg.at[b0 + b], idxg_v.at[pl.ds(slot * G, G)], semg[slot]).wait()
        pltpu.make_async_copy(idxs.at[b0 + b], idxs_v.at[pl.ds(slot * G, G)], semsd[slot]).wait()

    def fire_g(slot, j, rb):
        pltpu.async_copy(table.at[idxg_v.at[slot * G + j]], rows_v.at[rb], semr[rb])

    def wait_g(slot, j, rb):
        pltpu.make_async_copy(table.at[idxg_v.at[slot * G + j]], rows_v.at[rb], semr[rb]).wait()

    fire_idx(0, 0)
    fire_idx(1, 1)
    pltpu.sync_copy(zeros, acc.at[pl.ds(s * RT, RT)])
    plsc.subcore_barrier()
    wait_idx(0, 0)
    fire_g(0, 0, 0)
    fire_g(0, 1, 1)

    def body(i, carry):
        for b2 in range(2):
            b = 2 * i + b2
            slot, other = b2, 1 - b2
            for j in range(G):
                rb = j % 2
                wait_g(slot, j, rb)
                pltpu.sync_copy(rows_v.at[rb], acc.at[idxs_v.at[slot * G + j]], add=True)
                if j < G - 2:
                    fire_g(slot, j + 2, rb)
                if j == G - 3:
                    @pl.when(b + 1 < nb)
                    def _():
                        wait_idx(other, b + 1)
                if j >= G - 2:
                    @pl.when(b + 1 < nb)
                    def _():
                        fire_g(other, j - (G - 2), rb)
                if j == G - 1:
                    @pl.when(b + 2 < nb)
                    def _():
                        fire_idx(slot, b + 2)
        return carry

    lax.fori_loop(0, nb // 2, body, 0)
    plsc.subcore_barrier()
    pltpu.sync_copy(acc.at[pl.ds(s * RT, RT)], out.at[c, pl.ds(s * RT, RT)])


@functools.cache
def _sc_pass():
    return pl.kernel(
        _sc_pass_body,
        out_type=jax.ShapeDtypeStruct((NC, NP, D), jnp.float32),
        mesh=plsc.VectorSubcoreMesh(core_axis_name="c", subcore_axis_name="s",
                                    num_cores=NC, num_subcores=NS),
        scratch_types=[
            pltpu.VMEM((2 * G, KCH), jnp.int32),
            pltpu.VMEM((2 * G, KCH), jnp.int32),
            pltpu.VMEM((2, KCH, D), jnp.float32),
            pltpu.VMEM_SHARED((NP, D), jnp.float32),
            pltpu.SemaphoreType.DMA,
            pltpu.SemaphoreType.DMA,
            pltpu.SemaphoreType.DMA,
            pltpu.SemaphoreType.DMA,
            pltpu.SemaphoreType.DMA,
            pltpu.SemaphoreType.DMA,
        ],
    )


def _k1_body(x_ref, a0_ref, a1_ref, w1r_ref, w1n_ref, b1_ref, wp1_ref,
             bp1_ref, g1_ref, be1_ref, x1_ref, s1p_ref, logsm_ref):
    xb = x_ref[...]
    agg = a0_ref[0] + a1_ref[0]
    h = jnp.dot(agg, w1n_ref[...], preferred_element_type=jnp.float32)
    h = h + jnp.dot(xb, w1r_ref[...], preferred_element_type=jnp.float32)
    x1 = jnp.maximum(h + b1_ref[...], 0.0)
    x1_ref[...] = x1
    t = jnp.dot(x1, wp1_ref[...], preferred_element_type=jnp.float32) + bp1_ref[...]
    lane = lax.broadcasted_iota(jnp.int32, (RB, D), 1)
    valid = lane < C1
    mu = jnp.sum(jnp.where(valid, t, 0.0), axis=-1, keepdims=True) * (1.0 / C1)
    ctr = jnp.where(valid, t - mu, 0.0)
    var = jnp.sum(ctr * ctr, axis=-1, keepdims=True) * (1.0 / C1)
    ln = (t - mu) / jnp.sqrt(var + EPS) * g1_ref[...] + be1_ref[...]
    mx = jnp.max(jnp.where(valid, ln, -jnp.inf), axis=-1, keepdims=True)
    ex = jnp.where(valid, jnp.exp(ln - mx), 0.0)
    se = jnp.sum(ex, axis=-1, keepdims=True)
    logsm_ref[...] = (ln - mx - jnp.log(se))[:, :C1]
    s1 = ex / se
    s1p_ref[...] = jnp.where(lane == C1, 1.0, s1)


def _k1(x, aggp, w1rT, w1nT, b1r, wp1p, bp1p, g1p, be1p):
    blk = pl.BlockSpec((RB, D), lambda i: (i, 0))
    p0 = pl.BlockSpec((1, RB, D), lambda i: (0, i, 0))
    p1 = pl.BlockSpec((1, RB, D), lambda i: (1, i, 0))
    wblk = pl.BlockSpec((D, D), lambda i: (0, 0))
    vblk = pl.BlockSpec((1, D), lambda i: (0, 0))
    return pl.pallas_call(
        _k1_body,
        grid=(GB,),
        in_specs=[blk, p0, p1, wblk, wblk, vblk, wblk, vblk, vblk, vblk],
        out_specs=[blk, blk, pl.BlockSpec((RB, C1), lambda i: (i, 0))],
        out_shape=[
            jax.ShapeDtypeStruct((N, D), jnp.float32),
            jax.ShapeDtypeStruct((N, D), jnp.float32),
            jax.ShapeDtypeStruct((N, C1), jnp.float32),
        ],
    )(x, aggp, aggp, w1rT, w1nT, b1r, wp1p, bp1p, g1p, be1p)


_DN = (((0,), (0,)), ((), ()))


def _k2_body(x1_ref, s1p_ref, as0_ref, as1_ref, w2r_ref, w2n_ref, b2_ref,
             wp2_ref, bp2_ref, g2_ref, be2_ref,
             mc1_ref, o1_ref, mc2_ref, o2_ref, logsm2_ref,
             m1_ref, m2_ref, m3_ref, m4_ref):
    i = pl.program_id(0)

    @pl.when(i == 0)
    def _():
        m1_ref[...] = jnp.zeros((D, D), jnp.float32)
        m2_ref[...] = jnp.zeros((D, D), jnp.float32)
        m3_ref[...] = jnp.zeros((D, D), jnp.float32)
        m4_ref[...] = jnp.zeros((D, D), jnp.float32)

    s1p = s1p_ref[...]
    x1 = x1_ref[...]
    asf = as0_ref[0] + as1_ref[0]
    lane = lax.broadcasted_iota(jnp.int32, (RB, D), 1)
    deg = jnp.sum(jnp.where(lane == C1, asf, 0.0), axis=-1, keepdims=True)
    m1_ref[...] += lax.dot_general(s1p, x1, _DN, preferred_element_type=jnp.float32)
    m2_ref[...] += lax.dot_general(s1p, asf, _DN, preferred_element_type=jnp.float32)
    m3_ref[...] += lax.dot_general(s1p, s1p, _DN, preferred_element_type=jnp.float32)
    m4_ref[...] += lax.dot_general(s1p, s1p * deg, _DN, preferred_element_type=jnp.float32)

    @pl.when(i == GB - 1)
    def _():
        r = lax.broadcasted_iota(jnp.int32, (D, D), 0)
        c = lax.broadcasted_iota(jnp.int32, (D, D), 1)
        v100 = (r < C1) & (c < C1)
        eye100 = jnp.where((r == c) & (r < C1), 1.0, 0.0)
        eyeF = jnp.where(r == c, 1.0, 0.0)
        padj = jnp.where(v100, m2_ref[...], 0.0)
        num1 = jnp.sum(padj * eye100)
        den1 = jnp.sum(jnp.where(v100, m4_ref[...], 0.0) * eye100) + 1e-10
        mc1_ref[...] = jnp.reshape(-num1 / den1, (1, 1))
        sts1 = jnp.where(v100, m3_ref[...], 0.0)
        nf1 = jnp.sqrt(jnp.sum(sts1 * sts1))
        o1m = sts1 / (nf1 + 1e-10) - eye100 * 0.1
        o1_ref[...] = jnp.reshape(jnp.sqrt(jnp.sum(o1m * o1m)), (1, 1))
        adj = padj * (1.0 - eye100)
        dsum = jnp.sum(adj, axis=1, keepdims=True)
        dinv = 1.0 / (jnp.sqrt(dsum) + 1e-15)
        dinv_row = lax.dot_general(dinv, eyeF, _DN, preferred_element_type=jnp.float32)
        adjn = adj * dinv * dinv_row
        a_mask = jnp.where(adjn > jnp.float32(THR), 1.0, 0.0)
        px = jnp.where(r < C1, m1_ref[...], 0.0)
        agg2 = lax.dot_general(a_mask, px, _DN, preferred_element_type=jnp.float32)
        x2 = jnp.dot(agg2, w2n_ref[...], preferred_element_type=jnp.float32)
        x2 = x2 + jnp.dot(px, w2r_ref[...], preferred_element_type=jnp.float32)
        x2 = jnp.maximum(x2 + b2_ref[...], 0.0)
        x2 = jnp.where(r < C1, x2, 0.0)
        t2 = jnp.dot(x2, wp2_ref[...], preferred_element_type=jnp.float32) + bp2_ref[...]
        vK = c < C2
        mu2 = jnp.sum(jnp.where(vK, t2, 0.0), axis=-1, keepdims=True) * (1.0 / C2)
        ct2 = jnp.where(vK, t2 - mu2, 0.0)
        var2 = jnp.sum(ct2 * ct2, axis=-1, keepdims=True) * (1.0 / C2)
        ln2 = (t2 - mu2) / jnp.sqrt(var2 + EPS) * g2_ref[...] + be2_ref[...]
        mx2 = jnp.max(jnp.where(vK, ln2, -jnp.inf), axis=-1, keepdims=True)
        ex2 = jnp.where(vK, jnp.exp(ln2 - mx2), 0.0)
        se2 = jnp.sum(ex2, axis=-1, keepdims=True)
        logsm2_ref[...] = (ln2 - mx2 - jnp.log(se2))[:C1, :C2]
        s2 = jnp.where(r < C1, ex2 / se2, 0.0)
        as2 = jnp.dot(a_mask, s2, preferred_element_type=jnp.float32)
        deg2 = jnp.sum(a_mask, axis=-1, keepdims=True)
        padj2 = lax.dot_general(s2, as2, _DN, preferred_element_type=jnp.float32)
        eye10 = jnp.where((r == c) & (r < C2), 1.0, 0.0)
        num2 = jnp.sum(padj2 * eye10)
        den2 = jnp.sum(deg2 * jnp.sum(s2 * s2, axis=-1, keepdims=True)) + 1e-10
        mc2_ref[...] = jnp.reshape(-num2 / den2, (1, 1))
        sts2 = lax.dot_general(s2, s2, _DN, preferred_element_type=jnp.float32)
        nf2 = jnp.sqrt(jnp.sum(sts2 * sts2))
        o2m = sts2 / (nf2 + 1e-10) - eye10 * (1.0 / jnp.sqrt(jnp.float32(C2)))
        o2_ref[...] = jnp.reshape(jnp.sqrt(jnp.sum(o2m * o2m)), (1, 1))


def _k2(x1, s1p, asp, w2rT, w2nT, b2r, wp2p, bp2p, g2p, be2p):
    blk = pl.BlockSpec((RB, D), lambda i: (i, 0))
    p0 = pl.BlockSpec((1, RB, D), lambda i: (0, i, 0))
    p1 = pl.BlockSpec((1, RB, D), lambda i: (1, i, 0))
    wblk = pl.BlockSpec((D, D), lambda i: (0, 0))
    vblk = pl.BlockSpec((1, D), lambda i: (0, 0))
    sblk = pl.BlockSpec((1, 1), lambda i: (0, 0))
    return pl.pallas_call(
        _k2_body,
        grid=(GB,),
        in_specs=[blk, blk, p0, p1, wblk, wblk, vblk, wblk, vblk, vblk, vblk],
        out_specs=[sblk, sblk, sblk, sblk, pl.BlockSpec((C1, C2), lambda i: (0, 0))],
        out_shape=[
            jax.ShapeDtypeStruct((1, 1), jnp.float32),
            jax.ShapeDtypeStruct((1, 1), jnp.float32),
            jax.ShapeDtypeStruct((1, 1), jnp.float32),
            jax.ShapeDtypeStruct((1, 1), jnp.float32),
            jax.ShapeDtypeStruct((C1, C2), jnp.float32),
        ],
        scratch_shapes=[pltpu.VMEM((D, D), jnp.float32)] * 4,
    )(x1, s1p, asp, asp, w2rT, w2nT, b2r, wp2p, bp2p, g2p, be2p)


def kernel(x, edge_index, W1r, W1n, b1, Wp1, bp1, g1, be1,
           W2r, W2n, b2, Wp2, bp2, g2, be2):
    f32 = jnp.float32
    pad = EPAD - E
    pad0 = jnp.zeros((pad,), jnp.int32)
    padd = jnp.full((pad,), NP - 1, jnp.int32)
    src_g = jnp.concatenate([edge_index[0], pad0]).reshape(TB, G, KCH)
    src_s = jnp.concatenate([edge_index[0], padd]).reshape(TB, G, KCH)
    dst_g = jnp.concatenate([edge_index[1], pad0]).reshape(TB, G, KCH)
    dst_s = jnp.concatenate([edge_index[1], padd]).reshape(TB, G, KCH)
    zeros = jnp.zeros((RT, D), f32)

    aggp = _sc_pass()(x, src_g, dst_s, zeros)

    wp1p = jnp.zeros((D, D), f32).at[:, :C1].set(Wp1.T)
    bp1p = jnp.zeros((1, D), f32).at[0, :C1].set(bp1)
    g1p = jnp.zeros((1, D), f32).at[0, :C1].set(g1)
    be1p = jnp.zeros((1, D), f32).at[0, :C1].set(be1)
    x1, s1p, logsm1 = _k1(x, aggp, W1r.T, W1n.T,
                          b1.reshape(1, D), wp1p, bp1p, g1p, be1p)

    asp = _sc_pass()(s1p, dst_g, src_s, zeros)

    wp2p = jnp.zeros((D, D), f32).at[:, :C2].set(Wp2.T)
    bp2p = jnp.zeros((1, D), f32).at[0, :C2].set(bp2)
    g2p = jnp.zeros((1, D), f32).at[0, :C2].set(g2)
    be2p = jnp.zeros((1, D), f32).at[0, :C2].set(be2)
    mc1, o1, mc2, o2, logsm2 = _k2(x1, s1p, asp, W2r.T, W2n.T,
                                   b2.reshape(1, D), wp2p, bp2p, g2p, be2p)

    return (mc1[0, 0], o1[0, 0], mc2[0, 0], o2[0, 0], logsm1, logsm2)


# speedup vs baseline: 1.1146x; 1.1146x over previous
import functools

import jax
import jax.numpy as jnp
from jax import lax
from jax.experimental import pallas as pl
from jax.experimental.pallas import tpu as pltpu
from jax.experimental.pallas import tpu_sc as plsc

N = 10000
E = 320000
D = 128
C1 = 100
C2 = 10
THR = 1.0 / (C1 - 1)
EPS = 1e-5

NC = 2
NS = 16
KCH = 128
G = 8
NB0 = 16
NB1 = 4
TB = NS * (NB0 + NB1)
EPAD = TB * G * KCH
NP = 10240
RT = NP // NS

RB = 1000
GB = N // RB


def _sc_pass_body(table, idxg, idxs, zeros, out, idxg_v, idxs_v, rows_v, acc,
                  semg0, semg1, sems0, sems1, semr0, semr1):
    c = lax.axis_index("c")
    s = lax.axis_index("s")
    nb = jnp.where(c == 0, NB0, NB1)
    b0 = c * NS * NB0 + s * nb
    semg = (semg0, semg1)
    semsd = (sems0, sems1)
    semr = (semr0, semr1)

    def fire_idx(slot, b):
        pltpu.async_copy(idxg.at[b0 + b], idxg_v.at[pl.ds(slot * G, G)], semg[slot])
        pltpu.async_copy(idxs.at[b0 + b], idxs_v.at[pl.ds(slot * G, G)], semsd[slot])

    def wait_idx(slot, b):
        pltpu.make_async_copy(idxg.at[b0 + b], idxg_v.at[pl.ds(slot * G, G)], semg[slot]).wait()
        pltpu.make_async_copy(idxs.at[b0 + b], idxs_v.at[pl.ds(slot * G, G)], semsd[slot]).wait()

    def fire_g(slot, j, rb):
        pltpu.async_copy(table.at[idxg_v.at[slot * G + j]], rows_v.at[rb], semr[rb])

    def wait_g(slot, j, rb):
        pltpu.make_async_copy(table.at[idxg_v.at[slot * G + j]], rows_v.at[rb], semr[rb]).wait()

    fire_idx(0, 0)
    fire_idx(1, 1)
    pltpu.sync_copy(zeros, acc.at[pl.ds(s * RT, RT)])
    plsc.subcore_barrier()
    wait_idx(0, 0)
    fire_g(0, 0, 0)
    fire_g(0, 1, 1)

    def body(i, carry):
        for b2 in range(2):
            b = 2 * i + b2
            slot, other = b2, 1 - b2
            for j in range(G):
                rb = j % 2
                wait_g(slot, j, rb)
                pltpu.sync_copy(rows_v.at[rb], acc.at[idxs_v.at[slot * G + j]], add=True)
                if j < G - 2:
                    fire_g(slot, j + 2, rb)
                if j == G - 3:
                    @pl.when(b + 1 < nb)
                    def _():
                        wait_idx(other, b + 1)
                if j >= G - 2:
                    @pl.when(b + 1 < nb)
                    def _():
                        fire_g(other, j - (G - 2), rb)
                if j == G - 1:
                    @pl.when(b + 2 < nb)
                    def _():
                        fire_idx(slot, b + 2)
        return carry

    lax.fori_loop(0, nb // 2, body, 0)
    plsc.subcore_barrier()
    pltpu.sync_copy(acc.at[pl.ds(s * RT, RT)], out.at[c, pl.ds(s * RT, RT)])


@functools.cache
def _sc_pass():
    return pl.kernel(
        _sc_pass_body,
        out_type=jax.ShapeDtypeStruct((NC, NP, D), jnp.float32),
        mesh=plsc.VectorSubcoreMesh(core_axis_name="c", subcore_axis_name="s",
                                    num_cores=NC, num_subcores=NS),
        scratch_types=[
            pltpu.VMEM((2 * G, KCH), jnp.int32),
            pltpu.VMEM((2 * G, KCH), jnp.int32),
            pltpu.VMEM((2, KCH, D), jnp.float32),
            pltpu.VMEM_SHARED((NP, D), jnp.float32),
            pltpu.SemaphoreType.DMA,
            pltpu.SemaphoreType.DMA,
            pltpu.SemaphoreType.DMA,
            pltpu.SemaphoreType.DMA,
            pltpu.SemaphoreType.DMA,
            pltpu.SemaphoreType.DMA,
        ],
    )


def _k1_body(x_ref, a0_ref, a1_ref, w1r_ref, w1n_ref, b1_ref, wp1_ref,
             bp1_ref, g1_ref, be1_ref, x1_ref, s1p_ref, logsm_ref):
    xb = x_ref[...]
    agg = a0_ref[0] + a1_ref[0]
    h = jnp.dot(agg, w1n_ref[...], preferred_element_type=jnp.float32)
    h = h + jnp.dot(xb, w1r_ref[...], preferred_element_type=jnp.float32)
    x1 = jnp.maximum(h + b1_ref[...], 0.0)
    x1_ref[...] = x1
    t = jnp.dot(x1, wp1_ref[...], preferred_element_type=jnp.float32) + bp1_ref[...]
    lane = lax.broadcasted_iota(jnp.int32, (RB, D), 1)
    valid = lane < C1
    mu = jnp.sum(jnp.where(valid, t, 0.0), axis=-1, keepdims=True) * (1.0 / C1)
    ctr = jnp.where(valid, t - mu, 0.0)
    var = jnp.sum(ctr * ctr, axis=-1, keepdims=True) * (1.0 / C1)
    ln = (t - mu) / jnp.sqrt(var + EPS) * g1_ref[...] + be1_ref[...]
    mx = jnp.max(jnp.where(valid, ln, -jnp.inf), axis=-1, keepdims=True)
    ex = jnp.where(valid, jnp.exp(ln - mx), 0.0)
    se = jnp.sum(ex, axis=-1, keepdims=True)
    logsm_ref[...] = (ln - mx - jnp.log(se))[:, :C1]
    s1 = ex / se
    s1p_ref[...] = jnp.where(lane == C1, 1.0, s1)


def _k1(x, aggp, w1rT, w1nT, b1r, wp1p, bp1p, g1p, be1p):
    blk = pl.BlockSpec((RB, D), lambda i: (i, 0))
    p0 = pl.BlockSpec((1, RB, D), lambda i: (0, i, 0))
    p1 = pl.BlockSpec((1, RB, D), lambda i: (1, i, 0))
    wblk = pl.BlockSpec((D, D), lambda i: (0, 0))
    vblk = pl.BlockSpec((1, D), lambda i: (0, 0))
    return pl.pallas_call(
        _k1_body,
        grid=(GB,),
        in_specs=[blk, p0, p1, wblk, wblk, vblk, wblk, vblk, vblk, vblk],
        out_specs=[blk, blk, pl.BlockSpec((RB, C1), lambda i: (i, 0))],
        out_shape=[
            jax.ShapeDtypeStruct((N, D), jnp.float32),
            jax.ShapeDtypeStruct((N, D), jnp.float32),
            jax.ShapeDtypeStruct((N, C1), jnp.float32),
        ],
    )(x, aggp, aggp, w1rT, w1nT, b1r, wp1p, bp1p, g1p, be1p)


_DN = (((0,), (0,)), ((), ()))


def _k2_body(x1_ref, s1p_ref, as0_ref, as1_ref, w2r_ref, w2n_ref, b2_ref,
             wp2_ref, bp2_ref, g2_ref, be2_ref,
             mc1_ref, o1_ref, mc2_ref, o2_ref, logsm2_ref,
             m1_ref, m2_ref, m3_ref, m4_ref):
    i = pl.program_id(0)

    @pl.when(i == 0)
    def _():
        m1_ref[...] = jnp.zeros((D, D), jnp.float32)
        m2_ref[...] = jnp.zeros((D, D), jnp.float32)
        m3_ref[...] = jnp.zeros((D, D), jnp.float32)
        m4_ref[...] = jnp.zeros((D, D), jnp.float32)

    s1p = s1p_ref[...]
    x1 = x1_ref[...]
    asf = as0_ref[0] + as1_ref[0]
    lane = lax.broadcasted_iota(jnp.int32, (RB, D), 1)
    deg = jnp.sum(jnp.where(lane == C1, asf, 0.0), axis=-1, keepdims=True)
    m1_ref[...] += lax.dot_general(s1p, x1, _DN, preferred_element_type=jnp.float32)
    m2_ref[...] += lax.dot_general(s1p, asf, _DN, preferred_element_type=jnp.float32)
    m3_ref[...] += lax.dot_general(s1p, s1p, _DN, preferred_element_type=jnp.float32)
    m4_ref[...] += lax.dot_general(s1p, s1p * deg, _DN, preferred_element_type=jnp.float32)

    @pl.when(i == GB - 1)
    def _():
        r = lax.broadcasted_iota(jnp.int32, (D, D), 0)
        c = lax.broadcasted_iota(jnp.int32, (D, D), 1)
        v100 = (r < C1) & (c < C1)
        eye100 = jnp.where((r == c) & (r < C1), 1.0, 0.0)
        eyeF = jnp.where(r == c, 1.0, 0.0)
        padj = jnp.where(v100, m2_ref[...], 0.0)
        num1 = jnp.sum(padj * eye100)
        den1 = jnp.sum(jnp.where(v100, m4_ref[...], 0.0) * eye100) + 1e-10
        mc1_ref[...] = jnp.reshape(-num1 / den1, (1, 1))
        sts1 = jnp.where(v100, m3_ref[...], 0.0)
        nf1 = jnp.sqrt(jnp.sum(sts1 * sts1))
        o1m = sts1 / (nf1 + 1e-10) - eye100 * 0.1
        o1_ref[...] = jnp.reshape(jnp.sqrt(jnp.sum(o1m * o1m)), (1, 1))
        adj = padj * (1.0 - eye100)
        dsum = jnp.sum(adj, axis=1, keepdims=True)
        dinv = 1.0 / (jnp.sqrt(dsum) + 1e-15)
        dinv_row = lax.dot_general(dinv, eyeF, _DN, preferred_element_type=jnp.float32)
        adjn = adj * dinv * dinv_row
        a_mask = jnp.where(adjn > jnp.float32(THR), 1.0, 0.0)
        px = jnp.where(r < C1, m1_ref[...], 0.0)
        agg2 = lax.dot_general(a_mask, px, _DN, preferred_element_type=jnp.float32)
        x2 = jnp.dot(agg2, w2n_ref[...], preferred_element_type=jnp.float32)
        x2 = x2 + jnp.dot(px, w2r_ref[...], preferred_element_type=jnp.float32)
        x2 = jnp.maximum(x2 + b2_ref[...], 0.0)
        x2 = jnp.where(r < C1, x2, 0.0)
        t2 = jnp.dot(x2, wp2_ref[...], preferred_element_type=jnp.float32) + bp2_ref[...]
        vK = c < C2
        mu2 = jnp.sum(jnp.where(vK, t2, 0.0), axis=-1, keepdims=True) * (1.0 / C2)
        ct2 = jnp.where(vK, t2 - mu2, 0.0)
        var2 = jnp.sum(ct2 * ct2, axis=-1, keepdims=True) * (1.0 / C2)
        ln2 = (t2 - mu2) / jnp.sqrt(var2 + EPS) * g2_ref[...] + be2_ref[...]
        mx2 = jnp.max(jnp.where(vK, ln2, -jnp.inf), axis=-1, keepdims=True)
        ex2 = jnp.where(vK, jnp.exp(ln2 - mx2), 0.0)
        se2 = jnp.sum(ex2, axis=-1, keepdims=True)
        logsm2_ref[...] = (ln2 - mx2 - jnp.log(se2))[:C1, :C2]
        s2 = jnp.where(r < C1, ex2 / se2, 0.0)
        as2 = jnp.dot(a_mask, s2, preferred_element_type=jnp.float32)
        deg2 = jnp.sum(a_mask, axis=-1, keepdims=True)
        padj2 = lax.dot_general(s2, as2, _DN, preferred_element_type=jnp.float32)
        eye10 = jnp.where((r == c) & (r < C2), 1.0, 0.0)
        num2 = jnp.sum(padj2 * eye10)
        den2 = jnp.sum(deg2 * jnp.sum(s2 * s2, axis=-1, keepdims=True)) + 1e-10
        mc2_ref[...] = jnp.reshape(-num2 / den2, (1, 1))
        sts2 = lax.dot_general(s2, s2, _DN, preferred_element_type=jnp.float32)
        nf2 = jnp.sqrt(jnp.sum(sts2 * sts2))
        o2m = sts2 / (nf2 + 1e-10) - eye10 * (1.0 / jnp.sqrt(jnp.float32(C2)))
        o2_ref[...] = jnp.reshape(jnp.sqrt(jnp.sum(o2m * o2m)), (1, 1))


def _k2(x1, s1p, asp, w2rT, w2nT, b2r, wp2p, bp2p, g2p, be2p):
    blk = pl.BlockSpec((RB, D), lambda i: (i, 0))
    p0 = pl.BlockSpec((1, RB, D), lambda i: (0, i, 0))
    p1 = pl.BlockSpec((1, RB, D), lambda i: (1, i, 0))
    wblk = pl.BlockSpec((D, D), lambda i: (0, 0))
    vblk = pl.BlockSpec((1, D), lambda i: (0, 0))
    sblk = pl.BlockSpec((1, 1), lambda i: (0, 0))
    return pl.pallas_call(
        _k2_body,
        grid=(GB,),
        in_specs=[blk, blk, p0, p1, wblk, wblk, vblk, wblk, vblk, vblk, vblk],
        out_specs=[sblk, sblk, sblk, sblk, pl.BlockSpec((C1, C2), lambda i: (0, 0))],
        out_shape=[
            jax.ShapeDtypeStruct((1, 1), jnp.float32),
            jax.ShapeDtypeStruct((1, 1), jnp.float32),
            jax.ShapeDtypeStruct((1, 1), jnp.float32),
            jax.ShapeDtypeStruct((1, 1), jnp.float32),
            jax.ShapeDtypeStruct((C1, C2), jnp.float32),
        ],
        scratch_shapes=[pltpu.VMEM((D, D), jnp.float32)] * 4,
    )(x1, s1p, asp, asp, w2rT, w2nT, b2r, wp2p, bp2p, g2p, be2p)


def kernel(x, edge_index, W1r, W1n, b1, Wp1, bp1, g1, be1,
           W2r, W2n, b2, Wp2, bp2, g2, be2):
    f32 = jnp.float32
    pad = EPAD - E
    pad0 = jnp.zeros((pad,), jnp.int32)
    padd = jnp.full((pad,), NP - 1, jnp.int32)
    src_g = jnp.concatenate([edge_index[0], pad0]).reshape(TB, G, KCH)
    src_s = jnp.concatenate([edge_index[0], padd]).reshape(TB, G, KCH)
    dst_g = jnp.concatenate([edge_index[1], pad0]).reshape(TB, G, KCH)
    dst_s = jnp.concatenate([edge_index[1], padd]).reshape(TB, G, KCH)
    zeros = jnp.zeros((RT, D), f32)

    aggp = _sc_pass()(x, src_g, dst_s, zeros)

    wp1p = jnp.zeros((D, D), f32).at[:, :C1].set(Wp1.T)
    bp1p = jnp.zeros((1, D), f32).at[0, :C1].set(bp1)
    g1p = jnp.zeros((1, D), f32).at[0, :C1].set(g1)
    be1p = jnp.zeros((1, D), f32).at[0, :C1].set(be1)
    x1, s1p, logsm1 = _k1(x, aggp, W1r.T, W1n.T,
                          b1.reshape(1, D), wp1p, bp1p, g1p, be1p)

    asp = _sc_pass()(s1p, dst_g, src_s, zeros)

    wp2p = jnp.zeros((D, D), f32).at[:, :C2].set(Wp2.T)
    bp2p = jnp.zeros((1, D), f32).at[0, :C2].set(bp2)
    g2p = jnp.zeros((1, D), f32).at[0, :C2].set(g2)
    be2p = jnp.zeros((1, D), f32).at[0, :C2].set(be2)
    mc1, o1, mc2, o2, logsm2 = _k2(x1, s1p, asp, W2r.T, W2n.T,
                                   b2.reshape(1, D), wp2p, bp2p, g2p, be2p)

    return (mc1[0, 0], o1[0, 0], mc2[0, 0], o2[0, 0], logsm1, logsm2)
